# baseline (speedup 1.0000x reference)
import jax
import jax.numpy as jnp
from jax import lax
from jax.experimental import pallas as pl
from jax.experimental.pallas import tpu as pltpu
from jax.experimental.pallas import tpu_sc as plsc

N = 100000
D_X = 128
PE_DIM = 8
HID = 128
OUT = 16
G = 512

NW = 32
RPW = N // NW
CHUNK = 125
NCHUNK = RPW // CHUNK

BLK = 2000
NB = N // BLK


def _sc_segsum_body(x_hbm, b_hbm, out_hbm, rowbuf, idsbuf, acc):
    wid = lax.axis_index("c") * 16 + lax.axis_index("s")
    row0 = wid * RPW

    @pl.loop(0, G * D_X, step=16)
    def _zero(i):
        acc[pl.ds(i, 16)] = jnp.zeros((16,), jnp.float32)

    @pl.loop(0, NCHUNK)
    def _chunk(c):
        s = row0 + c * CHUNK
        pltpu.sync_copy(x_hbm.at[pl.ds(s * D_X, CHUNK * D_X)], rowbuf)
        base = (s // 8) * 8
        off = s - base
        pltpu.sync_copy(b_hbm.at[pl.ds(base, 136)], idsbuf.at[pl.ds(0, 136)])

        @pl.loop(0, CHUNK)
        def _row(i):
            seg = idsbuf[pl.ds(off + i, 16)][0]
            for k in range(D_X // 16):
                v = rowbuf[pl.ds(i * D_X + 16 * k, 16)]
                plsc.addupdate(acc.at[pl.ds(seg * D_X + 16 * k, 16)], v)

    pltpu.sync_copy(acc, out_hbm.at[pl.ds(wid * G * D_X, G * D_X)])


def _sc_segsum(x, b32):
    mesh = plsc.VectorSubcoreMesh(core_axis_name="c", subcore_axis_name="s")
    return pl.kernel(
        _sc_segsum_body,
        out_type=jax.ShapeDtypeStruct((NW * G * D_X,), jnp.float32),
        mesh=mesh,
        scratch_types=[
            pltpu.VMEM((CHUNK * D_X,), jnp.float32),
            pltpu.VMEM((152,), jnp.int32),
            pltpu.VMEM((G * D_X,), jnp.float32),
        ],
    )(x.reshape(N * D_X), b32)


def _tc_pe_counts_body(peb, bb, accp):
    step = pl.program_id(0)

    @pl.when(step == 0)
    def _init():
        accp[...] = jnp.zeros_like(accp)

    ids = bb[0, 0, :]
    seg = lax.broadcasted_iota(jnp.int32, (1, G), 1)
    onehot = (ids[:, None] == seg).astype(jnp.float32)
    pe1 = jnp.concatenate(
        [peb[...], jnp.ones((BLK, 1), jnp.float32)], axis=1)
    accp[...] += lax.dot_general(
        onehot, pe1, (((0,), (0,)), ((), ())),
        preferred_element_type=jnp.float32)


def _tc_pe_counts(pe, batch3):
    return pl.pallas_call(
        _tc_pe_counts_body,
        grid=(NB,),
        in_specs=[
            pl.BlockSpec((BLK, PE_DIM), lambda i: (i, 0)),
            pl.BlockSpec((1, 1, BLK), lambda i: (i, 0, 0)),
        ],
        out_specs=pl.BlockSpec((G, PE_DIM + 1), lambda i: (0, 0)),
        out_shape=jax.ShapeDtypeStruct((G, PE_DIM + 1), jnp.float32),
    )(pe, batch3)


def _tc_combine_body(parts, accp, W_enc, b_enc, W1, b1, W2, b2, out_ref):
    psum = jnp.sum(parts[...], axis=0)
    cnt = jnp.maximum(accp[:, PE_DIM:PE_DIM + 1], 1.0)
    pooled_x = psum / cnt
    pooled_pe = accp[:, :PE_DIM] / cnt
    h = (lax.dot_general(pooled_x, W_enc[:, :D_X],
                         (((1,), (1,)), ((), ())),
                         preferred_element_type=jnp.float32)
         + lax.dot_general(pooled_pe, W_enc[:, D_X:],
                           (((1,), (1,)), ((), ())),
                           preferred_element_type=jnp.float32)
         + b_enc[...])
    h1 = jnp.maximum(
        lax.dot_general(h, W1[...], (((1,), (1,)), ((), ())),
                        preferred_element_type=jnp.float32) + b1[...], 0.0)
    out_ref[...] = (
        lax.dot_general(h1, W2[...], (((1,), (1,)), ((), ())),
                        preferred_element_type=jnp.float32) + b2[...])


def _tc_combine(parts, accp, W_enc, b_enc, W1, b1, W2, b2):
    return pl.pallas_call(
        _tc_combine_body,
        out_shape=jax.ShapeDtypeStruct((G, OUT), jnp.float32),
    )(parts, accp, W_enc, b_enc.reshape(1, HID), W1,
      b1.reshape(1, HID), W2, b2.reshape(1, OUT))


def kernel(x, pe, batch, W_enc, b_enc, W1, b1, W2, b2):
    b32 = batch.astype(jnp.int32)
    parts = _sc_segsum(x, b32).reshape(NW, G, D_X)
    accp = _tc_pe_counts(pe, b32.reshape(NB, 1, BLK))
    return _tc_combine(parts, accp, W_enc, b_enc, W1, b1, W2, b2)


# speedup vs baseline: 1.8561x; 1.8561x over previous
import jax
import jax.numpy as jnp
from jax import lax
from jax.experimental import pallas as pl
from jax.experimental.pallas import tpu as pltpu
from jax.experimental.pallas import tpu_sc as plsc

N = 100000
D_X = 128
PE_DIM = 8
HID = 128
OUT = 16
G = 512

NW = 32
RPW = N // NW
CHUNK = 125
NCHUNK = RPW // CHUNK

BLK = 2000
NB = N // BLK


def _sc_segsum_body(x_hbm, b_hbm, out_hbm, rb0, rb1, ib0, ib1, acc,
                    sem0, sem1):
    wid = lax.axis_index("c") * 16 + lax.axis_index("s")
    row0 = wid * RPW

    @pl.loop(0, G * D_X, step=16)
    def _zero(i):
        acc[pl.ds(i, 16)] = jnp.zeros((16,), jnp.float32)

    def _start(c, rb, ib, sem):
        s = row0 + c * CHUNK
        pltpu.async_copy(x_hbm.at[pl.ds(s * D_X, CHUNK * D_X)], rb, sem)
        base = (s // 8) * 8
        pltpu.async_copy(b_hbm.at[pl.ds(base, 136)],
                         ib.at[pl.ds(0, 136)], sem)

    def _wait(c, rb, ib, sem):
        s = row0 + c * CHUNK
        pltpu.make_async_copy(
            x_hbm.at[pl.ds(s * D_X, CHUNK * D_X)], rb, sem).wait()
        base = (s // 8) * 8
        pltpu.make_async_copy(
            b_hbm.at[pl.ds(base, 136)], ib.at[pl.ds(0, 136)], sem).wait()

    def _process(c, rb, ib):
        s = row0 + c * CHUNK
        off = s - (s // 8) * 8

        @plsc.parallel_loop(0, CHUNK, 1, unroll=5)
        def _row(i):
            seg = ib[pl.ds(off + i, 16)][0]
            for k in range(D_X // 16):
                v = rb[pl.ds(i * D_X + 16 * k, 16)]
                plsc.addupdate(acc.at[pl.ds(seg * D_X + 16 * k, 16)], v)

    _start(0, rb0, ib0, sem0)

    @pl.loop(0, NCHUNK - 1, step=2)
    def _chunk(c):
        _start(c + 1, rb1, ib1, sem1)
        _wait(c, rb0, ib0, sem0)
        _process(c, rb0, ib0)
        _start(c + 2, rb0, ib0, sem0)
        _wait(c + 1, rb1, ib1, sem1)
        _process(c + 1, rb1, ib1)

    _wait(NCHUNK - 1, rb0, ib0, sem0)
    _process(NCHUNK - 1, rb0, ib0)

    pltpu.sync_copy(acc, out_hbm.at[pl.ds(wid * G * D_X, G * D_X)])


def _sc_segsum(x, b32):
    mesh = plsc.VectorSubcoreMesh(core_axis_name="c", subcore_axis_name="s")
    return pl.kernel(
        _sc_segsum_body,
        out_type=jax.ShapeDtypeStruct((NW * G * D_X,), jnp.float32),
        mesh=mesh,
        scratch_types=[
            pltpu.VMEM((CHUNK * D_X,), jnp.float32),
            pltpu.VMEM((CHUNK * D_X,), jnp.float32),
            pltpu.VMEM((152,), jnp.int32),
            pltpu.VMEM((152,), jnp.int32),
            pltpu.VMEM((G * D_X,), jnp.float32),
            pltpu.SemaphoreType.DMA,
            pltpu.SemaphoreType.DMA,
        ],
    )(x.reshape(N * D_X), b32)


def _tc_pe_counts_body(peb, bb, accp):
    step = pl.program_id(0)

    @pl.when(step == 0)
    def _init():
        accp[...] = jnp.zeros_like(accp)

    ids = bb[0, 0, :]
    seg = lax.broadcasted_iota(jnp.int32, (1, G), 1)
    onehot = (ids[:, None] == seg).astype(jnp.float32)
    pe1 = jnp.concatenate(
        [peb[...], jnp.ones((BLK, 1), jnp.float32)], axis=1)
    accp[...] += lax.dot_general(
        onehot, pe1, (((0,), (0,)), ((), ())),
        preferred_element_type=jnp.float32)


def _tc_pe_counts(pe, batch3):
    return pl.pallas_call(
        _tc_pe_counts_body,
        grid=(NB,),
        in_specs=[
            pl.BlockSpec((BLK, PE_DIM), lambda i: (i, 0)),
            pl.BlockSpec((1, 1, BLK), lambda i: (i, 0, 0)),
        ],
        out_specs=pl.BlockSpec((G, PE_DIM + 1), lambda i: (0, 0)),
        out_shape=jax.ShapeDtypeStruct((G, PE_DIM + 1), jnp.float32),
    )(pe, batch3)


def _tc_combine_body(parts, accp, W_enc, b_enc, W1, b1, W2, b2, out_ref):
    psum = jnp.sum(parts[...], axis=0)
    cnt = jnp.maximum(accp[:, PE_DIM:PE_DIM + 1], 1.0)
    pooled_x = psum / cnt
    pooled_pe = accp[:, :PE_DIM] / cnt
    h = (lax.dot_general(pooled_x, W_enc[:, :D_X],
                         (((1,), (1,)), ((), ())),
                         preferred_element_type=jnp.float32)
         + lax.dot_general(pooled_pe, W_enc[:, D_X:],
                           (((1,), (1,)), ((), ())),
                           preferred_element_type=jnp.float32)
         + b_enc[...])
    h1 = jnp.maximum(
        lax.dot_general(h, W1[...], (((1,), (1,)), ((), ())),
                        preferred_element_type=jnp.float32) + b1[...], 0.0)
    out_ref[...] = (
        lax.dot_general(h1, W2[...], (((1,), (1,)), ((), ())),
                        preferred_element_type=jnp.float32) + b2[...])


def _tc_combine(parts, accp, W_enc, b_enc, W1, b1, W2, b2):
    return pl.pallas_call(
        _tc_combine_body,
        out_shape=jax.ShapeDtypeStruct((G, OUT), jnp.float32),
    )(parts, accp, W_enc, b_enc.reshape(1, HID), W1,
      b1.reshape(1, HID), W2, b2.reshape(1, OUT))


def kernel(x, pe, batch, W_enc, b_enc, W1, b1, W2, b2):
    b32 = batch.astype(jnp.int32)
    parts = _sc_segsum(x, b32).reshape(NW, G, D_X)
    accp = _tc_pe_counts(pe, b32.reshape(NB, 1, BLK))
    return _tc_combine(parts, accp, W_enc, b_enc, W1, b1, W2, b2)
